# baseline (speedup 1.0000x reference)
import functools

import jax
import jax.numpy as jnp
from jax import lax
from jax.experimental import pallas as pl
from jax.experimental.pallas import tpu as pltpu
from jax.experimental.pallas import tpu_sc as plsc

EMBED_DIM = 64
_LANES = 16

_mesh = plsc.VectorSubcoreMesh(
    core_axis_name="c", subcore_axis_name="s", num_cores=1
)


@functools.partial(
    pl.kernel,
    mesh=_mesh,
    out_type=jax.ShapeDtypeStruct((1, EMBED_DIM), jnp.float32),
    scratch_types=[
        pltpu.VMEM((_LANES,), jnp.int32),
    ],
    compiler_params=pltpu.CompilerParams(use_tc_tiling_on_sc=True),
)
def _sc_lookup(ids_hbm, table_hbm, out_hbm, idx_v):
    s = lax.axis_index("s")

    @pl.when(s == 0)
    def _():
        pltpu.sync_copy(ids_hbm.at[0, pl.ds(0, _LANES)], idx_v)
        idx0 = idx_v[...][0]
        pltpu.sync_copy(table_hbm.at[pl.ds(idx0, 1), :], out_hbm)


def kernel(ids, table):
    return _sc_lookup(ids.astype(jnp.int32), table)[0]


# speedup vs baseline: 1.0569x; 1.0569x over previous
import jax
import jax.numpy as jnp
from jax import lax
from jax.experimental import pallas as pl
from jax.experimental.pallas import tpu as pltpu

EMBED_DIM = 64
_SUB = 8


def _lookup_body(idx_ref, table_ref, out_ref):
    r = idx_ref[0] % _SUB
    row = table_ref[pl.ds(r, 1), :]
    out_ref[...] = jnp.broadcast_to(row, (_SUB, EMBED_DIM))


def kernel(ids, table):
    sidx = lax.slice(ids, (0, 0), (1, 1)).reshape((1,)).astype(jnp.int32)
    grid_spec = pltpu.PrefetchScalarGridSpec(
        num_scalar_prefetch=1,
        grid=(1,),
        in_specs=[
            pl.BlockSpec((_SUB, EMBED_DIM), lambda i, idx_ref: (idx_ref[0] // _SUB, 0)),
        ],
        out_specs=pl.BlockSpec((_SUB, EMBED_DIM), lambda i, idx_ref: (0, 0)),
    )
    out = pl.pallas_call(
        _lookup_body,
        grid_spec=grid_spec,
        out_shape=jax.ShapeDtypeStruct((_SUB, EMBED_DIM), jnp.float32),
    )(sidx, table)
    return out[0]


# speedup vs baseline: 73.9341x; 69.9547x over previous
import jax
import jax.numpy as jnp
from jax import lax
from jax.experimental import pallas as pl
from jax.experimental.pallas import tpu as pltpu

EMBED_DIM = 64
_LANE = 128


def _lookup_body(idx_ref, tblock_ref, out_ref):
    c = idx_ref[0] % _LANE
    lanes = lax.broadcasted_iota(jnp.int32, (EMBED_DIM, _LANE), 1)
    sel = jnp.where(lanes == c, tblock_ref[...], 0.0)
    out_ref[...] = jnp.sum(sel, axis=1, keepdims=True)


def kernel(ids, table):
    sidx = lax.slice(ids, (0, 0), (1, 1)).reshape((1,)).astype(jnp.int32)
    grid_spec = pltpu.PrefetchScalarGridSpec(
        num_scalar_prefetch=1,
        grid=(1,),
        in_specs=[
            pl.BlockSpec((EMBED_DIM, _LANE), lambda i, idx_ref: (0, idx_ref[0] // _LANE)),
        ],
        out_specs=pl.BlockSpec((EMBED_DIM, 1), lambda i, idx_ref: (0, 0)),
    )
    out = pl.pallas_call(
        _lookup_body,
        grid_spec=grid_spec,
        out_shape=jax.ShapeDtypeStruct((EMBED_DIM, 1), jnp.float32),
    )(sidx, table.T)
    return out[:, 0]
